# baseline (speedup 1.0000x reference)
import jax
import jax.numpy as jnp
from jax import lax
from jax.experimental import pallas as pl
from jax.experimental.pallas import tpu as pltpu, tpu_sc as plsc

N_WORKERS = 32
CHUNK = 4096
W = 128
GROUPS = CHUNK // 16
ROWS = CHUNK // W


def _body(xyz_hbm, table_hbm, colors_hbm, dens_hbm,
          xyz_v, gidx_v, sub_v, cond_v, rows_v, outc_v, outd_v, sem):
    wid = lax.axis_index("s") * 2 + lax.axis_index("c")
    n_pts = dens_hbm.shape[0]
    per_worker = n_pts // N_WORKERS
    n_chunks = per_worker // CHUNK

    ii = lax.iota(jnp.int32, 16)
    i3 = ii * 3
    zero16 = jnp.zeros((16,), jnp.int32)

    def chunk_body(k, _):
        base = wid * per_worker + k * CHUNK

        pltpu.sync_copy(xyz_hbm.at[pl.ds(base * 3, CHUNK * 3)], xyz_v)

        def index_body(g, _):
            b3 = g * 48
            x = plsc.load_gather(xyz_v, [i3 + b3])
            y = plsc.load_gather(xyz_v, [i3 + (b3 + 1)])
            z = plsc.load_gather(xyz_v, [i3 + (b3 + 2)])
            ux = jnp.clip(x * 128.0 + 128.0, 0.0, 255.0).astype(jnp.int32)
            uy = jnp.clip(y * 128.0 + 128.0, 0.0, 255.0).astype(jnp.int32)
            uz = jnp.clip(z * 128.0 + 128.0, 0.0, 255.0).astype(jnp.int32)
            flat = (ux << 16) | (uy << 8) | uz
            m = jnp.maximum(jnp.maximum(jnp.abs(x), jnp.abs(y)), jnp.abs(z))
            condf = jnp.where(m < 1.0, 1.0, 0.0)
            r = g // 8
            c0 = (g % 8) * 16
            gidx_v[r, pl.ds(c0, 16)] = flat >> 2
            sub_v[pl.ds(g * 16, 16)] = (flat & 3) * 4
            cond_v[pl.ds(g * 16, 16)] = condf
            return 0

        lax.fori_loop(0, GROUPS, index_body, 0)

        def fire(r, _):
            pltpu.async_copy(table_hbm.at[gidx_v.at[r]], rows_v.at[r], sem)
            return 0

        def drain(r, _):
            pltpu.make_async_copy(
                table_hbm.at[gidx_v.at[r]], rows_v.at[r], sem).wait()
            return 0

        lax.fori_loop(0, ROWS, fire, 0)
        lax.fori_loop(0, ROWS, drain, 0)

        def post_body(g, _):
            r = g // 8
            c0 = (g % 8) * 16
            cvec = ii + c0
            rs = zero16 + r
            sub = sub_v[pl.ds(g * 16, 16)]
            vr = plsc.load_gather(rows_v, [rs, cvec, sub])
            vg = plsc.load_gather(rows_v, [rs, cvec, sub + 1])
            vb = plsc.load_gather(rows_v, [rs, cvec, sub + 2])
            vd = plsc.load_gather(rows_v, [rs, cvec, sub + 3])
            cf = cond_v[pl.ds(g * 16, 16)]
            sr = 1.0 / (1.0 + jnp.exp(-(vr * cf)))
            sg = 1.0 / (1.0 + jnp.exp(-(vg * cf)))
            sb = 1.0 / (1.0 + jnp.exp(-(vb * cf)))
            dd = jnp.maximum(vd * cf * 10.0, 0.0)
            p3 = i3 + g * 48
            plsc.store_scatter(outc_v, [p3], sr)
            plsc.store_scatter(outc_v, [p3 + 1], sg)
            plsc.store_scatter(outc_v, [p3 + 2], sb)
            outd_v[pl.ds(g * 16, 16)] = dd
            return 0

        lax.fori_loop(0, GROUPS, post_body, 0)

        pltpu.sync_copy(outc_v, colors_hbm.at[pl.ds(base * 3, CHUNK * 3)])
        pltpu.sync_copy(outd_v, dens_hbm.at[pl.ds(base, CHUNK)])
        return 0

    lax.fori_loop(0, n_chunks, chunk_body, 0)


@jax.jit
def _sc_voxels(xyz_flat, table):
    n_pts = xyz_flat.shape[0] // 3
    mesh = plsc.VectorSubcoreMesh(core_axis_name="c", subcore_axis_name="s")
    return pl.kernel(
        _body,
        out_type=(
            jax.ShapeDtypeStruct((n_pts * 3,), jnp.float32),
            jax.ShapeDtypeStruct((n_pts,), jnp.float32),
        ),
        mesh=mesh,
        compiler_params=pltpu.CompilerParams(
            needs_layout_passes=False, use_tc_tiling_on_sc=False),
        scratch_types=[
            pltpu.VMEM((CHUNK * 3,), jnp.float32),
            pltpu.VMEM((ROWS, W), jnp.int32),
            pltpu.VMEM((CHUNK,), jnp.int32),
            pltpu.VMEM((CHUNK,), jnp.float32),
            pltpu.VMEM((ROWS, W, 16), jnp.float32),
            pltpu.VMEM((CHUNK * 3,), jnp.float32),
            pltpu.VMEM((CHUNK,), jnp.float32),
            pltpu.SemaphoreType.DMA,
        ],
    )(xyz_flat, table)


def kernel(xyz, voxels):
    n = xyz.shape[0]
    colors_flat, dens = _sc_voxels(
        xyz.reshape(-1), voxels.reshape(-1, 16))
    return colors_flat.reshape(n, 3), dens


# speedup vs baseline: 58.1296x; 58.1296x over previous
import jax
import jax.numpy as jnp
from jax import lax
from jax.experimental import pallas as pl
from jax.experimental.pallas import tpu as pltpu, tpu_sc as plsc

N_WORKERS = 32
CHUNK = 1024
W = 128
GROUPS = CHUNK // 16
RPC = CHUNK // W
STREAMS = 4 * RPC


def _body(xs_hbm, ys_hbm, zs_hbm, tab_hbm, ro_hbm, go_hbm, bo_hbm, do_hbm,
          xs_v, ys_v, zs_v, gidx_v, sub_v, cond_v, rows_v,
          ro_v, go_v, bo_v, do_v, sem):
    wid = lax.axis_index("s") * 2 + lax.axis_index("c")
    n_pts = xs_hbm.shape[0]
    per_worker = n_pts // N_WORKERS
    n_chunks = per_worker // CHUNK

    ii = lax.iota(jnp.int32, 16)

    def chunk_body(k, _):
        base = wid * per_worker + k * CHUNK

        pltpu.sync_copy(xs_hbm.at[pl.ds(base, CHUNK)], xs_v)
        pltpu.sync_copy(ys_hbm.at[pl.ds(base, CHUNK)], ys_v)
        pltpu.sync_copy(zs_hbm.at[pl.ds(base, CHUNK)], zs_v)

        def index_body(g, _):
            o = g * 16
            x = xs_v[pl.ds(o, 16)]
            y = ys_v[pl.ds(o, 16)]
            z = zs_v[pl.ds(o, 16)]
            ux = jnp.clip(x * 128.0 + 128.0, 0.0, 255.0).astype(jnp.int32)
            uy = jnp.clip(y * 128.0 + 128.0, 0.0, 255.0).astype(jnp.int32)
            uz = jnp.clip(z * 128.0 + 128.0, 0.0, 255.0).astype(jnp.int32)
            zt = uz >> 7
            zl = uz & 127
            g0 = ((((ux << 8) | uy) << 1 | zt) << 5) | (zl >> 4)
            m = jnp.maximum(jnp.maximum(jnp.abs(x), jnp.abs(y)), jnp.abs(z))
            condf = jnp.where(m < 1.0, 1.0, 0.0)
            r = g // 8
            c0 = (g % 8) * 16
            gidx_v[r, pl.ds(c0, 16)] = g0
            gidx_v[RPC + r, pl.ds(c0, 16)] = g0 + 8
            gidx_v[2 * RPC + r, pl.ds(c0, 16)] = g0 + 16
            gidx_v[3 * RPC + r, pl.ds(c0, 16)] = g0 + 24
            sub_v[pl.ds(o, 16)] = zl & 15
            cond_v[pl.ds(o, 16)] = condf
            return 0

        lax.fori_loop(0, GROUPS, index_body, 0)

        def fire(r, _):
            pltpu.async_copy(tab_hbm.at[gidx_v.at[r]], rows_v.at[r], sem)
            return 0

        def drain(r, _):
            pltpu.make_async_copy(
                tab_hbm.at[gidx_v.at[r]], rows_v.at[r], sem).wait()
            return 0

        lax.fori_loop(0, STREAMS, fire, 0)
        lax.fori_loop(0, STREAMS, drain, 0)

        def post_body(g, _):
            o = g * 16
            r = g // 8
            cvec = ii + (g % 8) * 16
            sub = sub_v[pl.ds(o, 16)]
            cf = cond_v[pl.ds(o, 16)]
            vr = plsc.load_gather(rows_v, [jnp.full((16,), r, jnp.int32), cvec, sub])
            vg = plsc.load_gather(rows_v, [jnp.full((16,), RPC + r, jnp.int32), cvec, sub])
            vb = plsc.load_gather(rows_v, [jnp.full((16,), 2 * RPC + r, jnp.int32), cvec, sub])
            vd = plsc.load_gather(rows_v, [jnp.full((16,), 3 * RPC + r, jnp.int32), cvec, sub])
            ro_v[pl.ds(o, 16)] = 1.0 / (1.0 + jnp.exp(-(vr * cf)))
            go_v[pl.ds(o, 16)] = 1.0 / (1.0 + jnp.exp(-(vg * cf)))
            bo_v[pl.ds(o, 16)] = 1.0 / (1.0 + jnp.exp(-(vb * cf)))
            do_v[pl.ds(o, 16)] = jnp.maximum(vd * cf * 10.0, 0.0)
            return 0

        lax.fori_loop(0, GROUPS, post_body, 0)

        pltpu.sync_copy(ro_v, ro_hbm.at[pl.ds(base, CHUNK)])
        pltpu.sync_copy(go_v, go_hbm.at[pl.ds(base, CHUNK)])
        pltpu.sync_copy(bo_v, bo_hbm.at[pl.ds(base, CHUNK)])
        pltpu.sync_copy(do_v, do_hbm.at[pl.ds(base, CHUNK)])
        return 0

    lax.fori_loop(0, n_chunks, chunk_body, 0)


@jax.jit
def _sc_voxels(xs, ys, zs, tab):
    n_pts = xs.shape[0]
    mesh = plsc.VectorSubcoreMesh(core_axis_name="c", subcore_axis_name="s")
    return pl.kernel(
        _body,
        out_type=(
            jax.ShapeDtypeStruct((n_pts,), jnp.float32),
            jax.ShapeDtypeStruct((n_pts,), jnp.float32),
            jax.ShapeDtypeStruct((n_pts,), jnp.float32),
            jax.ShapeDtypeStruct((n_pts,), jnp.float32),
        ),
        mesh=mesh,
        compiler_params=pltpu.CompilerParams(
            needs_layout_passes=False, use_tc_tiling_on_sc=False),
        scratch_types=[
            pltpu.VMEM((CHUNK,), jnp.float32),
            pltpu.VMEM((CHUNK,), jnp.float32),
            pltpu.VMEM((CHUNK,), jnp.float32),
            pltpu.VMEM((STREAMS, W), jnp.int32),
            pltpu.VMEM((CHUNK,), jnp.int32),
            pltpu.VMEM((CHUNK,), jnp.float32),
            pltpu.VMEM((STREAMS, W, 16), jnp.float32),
            pltpu.VMEM((CHUNK,), jnp.float32),
            pltpu.VMEM((CHUNK,), jnp.float32),
            pltpu.VMEM((CHUNK,), jnp.float32),
            pltpu.VMEM((CHUNK,), jnp.float32),
            pltpu.SemaphoreType.DMA,
        ],
    )(xs, ys, zs, tab)


def kernel(xyz, voxels):
    tab = (voxels.reshape(256, 256, 2, 128, 4)
           .transpose(0, 1, 2, 4, 3).reshape(1 << 22, 16))
    r, g, b, d = _sc_voxels(xyz[:, 0], xyz[:, 1], xyz[:, 2], tab)
    return jnp.stack([r, g, b], axis=1), d


# speedup vs baseline: 92.5541x; 1.5922x over previous
import jax
import jax.numpy as jnp
from jax import lax
from jax.experimental import pallas as pl
from jax.experimental.pallas import tpu as pltpu, tpu_sc as plsc

N_WORKERS = 32
CHUNK = 1024
W = 128
GROUPS = CHUNK // 16
RPC = CHUNK // W + 1


def _body(xs_hbm, ys_hbm, zs_hbm, tab_hbm, ro_hbm, go_hbm, bo_hbm, do_hbm,
          xs_v, ys_v, zs_v, cidx_v, sub_v, pos_v, rows_v,
          ro_v, go_v, bo_v, do_v, sem):
    wid = lax.axis_index("s") * 2 + lax.axis_index("c")
    n_pts = xs_hbm.shape[0]
    per_worker = n_pts // N_WORKERS
    n_chunks = per_worker // CHUNK

    ii = lax.iota(jnp.int32, 16)
    half16 = jnp.full((16,), 0.5, jnp.float32)
    zerof16 = jnp.zeros((16,), jnp.float32)

    def chunk_body(k, _):
        base = wid * per_worker + k * CHUNK

        pltpu.sync_copy(xs_hbm.at[pl.ds(base, CHUNK)], xs_v)
        pltpu.sync_copy(ys_hbm.at[pl.ds(base, CHUNK)], ys_v)
        pltpu.sync_copy(zs_hbm.at[pl.ds(base, CHUNK)], zs_v)

        def index_body(g, mcount):
            o = g * 16
            x = xs_v[pl.ds(o, 16)]
            y = ys_v[pl.ds(o, 16)]
            z = zs_v[pl.ds(o, 16)]
            ux = jnp.clip(x * 128.0 + 128.0, 0.0, 255.0).astype(jnp.int32)
            uy = jnp.clip(y * 128.0 + 128.0, 0.0, 255.0).astype(jnp.int32)
            uz = jnp.clip(z * 128.0 + 128.0, 0.0, 255.0).astype(jnp.int32)
            zt = uz >> 7
            zl = uz & 127
            g0 = ((((ux << 8) | uy) << 1 | zt) << 5) | (zl >> 4)
            m = jnp.maximum(jnp.maximum(jnp.abs(x), jnp.abs(y)), jnp.abs(z))
            keep = m < 1.0
            tgt = mcount + plsc.cumsum(keep.astype(jnp.int32)) - 1
            row = tgt >> 7
            col = tgt & 127
            plsc.store_scatter(cidx_v, [row, col], g0, mask=keep)
            plsc.store_scatter(cidx_v, [RPC + row, col], g0 + 8, mask=keep)
            plsc.store_scatter(cidx_v, [2 * RPC + row, col], g0 + 16, mask=keep)
            plsc.store_scatter(cidx_v, [3 * RPC + row, col], g0 + 24, mask=keep)
            plsc.store_scatter(sub_v, [tgt], zl & 15, mask=keep)
            plsc.store_scatter(pos_v, [tgt], ii + o, mask=keep)
            ro_v[pl.ds(o, 16)] = half16
            go_v[pl.ds(o, 16)] = half16
            bo_v[pl.ds(o, 16)] = half16
            do_v[pl.ds(o, 16)] = zerof16
            return mcount + jnp.max(
                plsc.all_reduce_population_count(keep))

        mcount = lax.fori_loop(0, GROUPS, index_body, jnp.int32(0))

        for j in range(8):
            tgt = mcount + j * 16 + ii
            row = tgt >> 7
            col = tgt & 127
            dummy = ((wid << 8) | (j * 16 + ii)) << 5
            plsc.store_scatter(cidx_v, [row, col], dummy)
            plsc.store_scatter(cidx_v, [RPC + row, col], dummy)
            plsc.store_scatter(cidx_v, [2 * RPC + row, col], dummy)
            plsc.store_scatter(cidx_v, [3 * RPC + row, col], dummy)

        n_streams = (mcount + 127) >> 7

        def fire(r, _):
            for c in range(4):
                pltpu.async_copy(tab_hbm.at[cidx_v.at[c * RPC + r]],
                                 rows_v.at[c * RPC + r], sem)
            return 0

        def drain(r, _):
            for c in range(4):
                pltpu.make_async_copy(tab_hbm.at[cidx_v.at[c * RPC + r]],
                                      rows_v.at[c * RPC + r], sem).wait()
            return 0

        lax.fori_loop(0, n_streams, fire, 0)
        lax.fori_loop(0, n_streams, drain, 0)

        def post_body(t, _):
            o = t * 16
            s = o + ii
            active = s < mcount
            sub = sub_v[pl.ds(o, 16)]
            p = pos_v[pl.ds(o, 16)]
            row = s >> 7
            col = s & 127
            vr = plsc.load_gather(rows_v, [row, col, sub], mask=active)
            vg = plsc.load_gather(rows_v, [RPC + row, col, sub], mask=active)
            vb = plsc.load_gather(rows_v, [2 * RPC + row, col, sub], mask=active)
            vd = plsc.load_gather(rows_v, [3 * RPC + row, col, sub], mask=active)
            plsc.store_scatter(ro_v, [p], 1.0 / (1.0 + jnp.exp(-vr)), mask=active)
            plsc.store_scatter(go_v, [p], 1.0 / (1.0 + jnp.exp(-vg)), mask=active)
            plsc.store_scatter(bo_v, [p], 1.0 / (1.0 + jnp.exp(-vb)), mask=active)
            plsc.store_scatter(do_v, [p], jnp.maximum(vd * 10.0, 0.0), mask=active)
            return 0

        lax.fori_loop(0, (mcount + 15) >> 4, post_body, 0)

        pltpu.sync_copy(ro_v, ro_hbm.at[pl.ds(base, CHUNK)])
        pltpu.sync_copy(go_v, go_hbm.at[pl.ds(base, CHUNK)])
        pltpu.sync_copy(bo_v, bo_hbm.at[pl.ds(base, CHUNK)])
        pltpu.sync_copy(do_v, do_hbm.at[pl.ds(base, CHUNK)])
        return 0

    lax.fori_loop(0, n_chunks, chunk_body, 0)


@jax.jit
def _sc_voxels(xs, ys, zs, tab):
    n_pts = xs.shape[0]
    mesh = plsc.VectorSubcoreMesh(core_axis_name="c", subcore_axis_name="s")
    return pl.kernel(
        _body,
        out_type=(
            jax.ShapeDtypeStruct((n_pts,), jnp.float32),
            jax.ShapeDtypeStruct((n_pts,), jnp.float32),
            jax.ShapeDtypeStruct((n_pts,), jnp.float32),
            jax.ShapeDtypeStruct((n_pts,), jnp.float32),
        ),
        mesh=mesh,
        compiler_params=pltpu.CompilerParams(
            needs_layout_passes=False, use_tc_tiling_on_sc=False),
        scratch_types=[
            pltpu.VMEM((CHUNK,), jnp.float32),
            pltpu.VMEM((CHUNK,), jnp.float32),
            pltpu.VMEM((CHUNK,), jnp.float32),
            pltpu.VMEM((4 * RPC, W), jnp.int32),
            pltpu.VMEM((CHUNK + 128,), jnp.int32),
            pltpu.VMEM((CHUNK + 128,), jnp.int32),
            pltpu.VMEM((4 * RPC, W, 16), jnp.float32),
            pltpu.VMEM((CHUNK,), jnp.float32),
            pltpu.VMEM((CHUNK,), jnp.float32),
            pltpu.VMEM((CHUNK,), jnp.float32),
            pltpu.VMEM((CHUNK,), jnp.float32),
            pltpu.SemaphoreType.DMA,
        ],
    )(xs, ys, zs, tab)


def kernel(xyz, voxels):
    tab = (voxels.reshape(256, 256, 2, 128, 4)
           .transpose(0, 1, 2, 4, 3).reshape(1 << 22, 16))
    r, g, b, d = _sc_voxels(xyz[:, 0], xyz[:, 1], xyz[:, 2], tab)
    return jnp.stack([r, g, b], axis=1), d


# speedup vs baseline: 120.7872x; 1.3050x over previous
import jax
import jax.numpy as jnp
from jax import lax
from jax.experimental import pallas as pl
from jax.experimental.pallas import tpu as pltpu, tpu_sc as plsc

N_WORKERS = 32
CHUNK = 1024
W = 128
GROUPS = CHUNK // 16
RPC = CHUNK // W + 1


def _body(xs_hbm, ys_hbm, zs_hbm, tab_hbm, ro_hbm, go_hbm, bo_hbm, do_hbm,
          xs_v, ys_v, zs_v,
          cidx_a, sub_a, pos_a, out_a,
          cidx_b, sub_b, pos_b, out_b,
          rows_v, sem_a, sem_b):
    wid = lax.axis_index("s") * 2 + lax.axis_index("c")
    n_pts = xs_hbm.shape[0]
    per_worker = n_pts // N_WORKERS
    n_chunks = per_worker // CHUNK

    ii = lax.iota(jnp.int32, 16)
    half16 = jnp.full((16,), 0.5, jnp.float32)
    zerof16 = jnp.zeros((16,), jnp.float32)

    def stage_a(k, cidx_v, sub_v, pos_v, out_v):
        base = wid * per_worker + k * CHUNK
        pltpu.sync_copy(xs_hbm.at[pl.ds(base, CHUNK)], xs_v)
        pltpu.sync_copy(ys_hbm.at[pl.ds(base, CHUNK)], ys_v)
        pltpu.sync_copy(zs_hbm.at[pl.ds(base, CHUNK)], zs_v)

        def index_body(g, mcount):
            o = g * 16
            x = xs_v[pl.ds(o, 16)]
            y = ys_v[pl.ds(o, 16)]
            z = zs_v[pl.ds(o, 16)]
            ux = jnp.clip(x * 128.0 + 128.0, 0.0, 255.0).astype(jnp.int32)
            uy = jnp.clip(y * 128.0 + 128.0, 0.0, 255.0).astype(jnp.int32)
            uz = jnp.clip(z * 128.0 + 128.0, 0.0, 255.0).astype(jnp.int32)
            zt = uz >> 7
            zl = uz & 127
            g0 = ((((ux << 8) | uy) << 1 | zt) << 5) | (zl >> 4)
            m = jnp.maximum(jnp.maximum(jnp.abs(x), jnp.abs(y)), jnp.abs(z))
            keep = m < 1.0
            tgt = mcount + plsc.cumsum(keep.astype(jnp.int32)) - 1
            row = tgt >> 7
            col = tgt & 127
            plsc.store_scatter(cidx_v, [row, col], g0, mask=keep)
            plsc.store_scatter(cidx_v, [RPC + row, col], g0 + 8, mask=keep)
            plsc.store_scatter(cidx_v, [2 * RPC + row, col], g0 + 16, mask=keep)
            plsc.store_scatter(cidx_v, [3 * RPC + row, col], g0 + 24, mask=keep)
            plsc.store_scatter(sub_v, [tgt], zl & 15, mask=keep)
            plsc.store_scatter(pos_v, [tgt], ii + o, mask=keep)
            out_v[0, pl.ds(o, 16)] = half16
            out_v[1, pl.ds(o, 16)] = half16
            out_v[2, pl.ds(o, 16)] = half16
            out_v[3, pl.ds(o, 16)] = zerof16
            return mcount + jnp.max(plsc.all_reduce_population_count(keep))

        mcount = lax.fori_loop(0, GROUPS, index_body, jnp.int32(0))

        for j in range(8):
            tgt = mcount + j * 16 + ii
            row = tgt >> 7
            col = tgt & 127
            dummy = ((wid << 8) | (j * 16 + ii)) << 5
            plsc.store_scatter(cidx_v, [row, col], dummy)
            plsc.store_scatter(cidx_v, [RPC + row, col], dummy)
            plsc.store_scatter(cidx_v, [2 * RPC + row, col], dummy)
            plsc.store_scatter(cidx_v, [3 * RPC + row, col], dummy)
        return mcount

    def fire(mcount, cidx_v, sem):
        def fire_body(r, _):
            for c in range(4):
                pltpu.async_copy(tab_hbm.at[cidx_v.at[c * RPC + r]],
                                 rows_v.at[c * RPC + r], sem)
            return 0
        lax.fori_loop(0, (mcount + 127) >> 7, fire_body, 0)

    def drain(mcount, cidx_v, sem):
        def drain_body(r, _):
            for c in range(4):
                pltpu.make_async_copy(tab_hbm.at[cidx_v.at[c * RPC + r]],
                                      rows_v.at[c * RPC + r], sem).wait()
            return 0
        lax.fori_loop(0, (mcount + 127) >> 7, drain_body, 0)

    def stage_c(k, mcount, sub_v, pos_v, out_v):
        def post_body(t, _):
            o = t * 16
            s = o + ii
            active = s < mcount
            sub = sub_v[pl.ds(o, 16)]
            p = pos_v[pl.ds(o, 16)]
            row = s >> 7
            col = s & 127
            vr = plsc.load_gather(rows_v, [row, col, sub], mask=active)
            vg = plsc.load_gather(rows_v, [RPC + row, col, sub], mask=active)
            vb = plsc.load_gather(rows_v, [2 * RPC + row, col, sub], mask=active)
            vd = plsc.load_gather(rows_v, [3 * RPC + row, col, sub], mask=active)
            plsc.store_scatter(out_v, [jnp.zeros((16,), jnp.int32), p],
                               1.0 / (1.0 + jnp.exp(-vr)), mask=active)
            plsc.store_scatter(out_v, [jnp.full((16,), 1, jnp.int32), p],
                               1.0 / (1.0 + jnp.exp(-vg)), mask=active)
            plsc.store_scatter(out_v, [jnp.full((16,), 2, jnp.int32), p],
                               1.0 / (1.0 + jnp.exp(-vb)), mask=active)
            plsc.store_scatter(out_v, [jnp.full((16,), 3, jnp.int32), p],
                               jnp.maximum(vd * 10.0, 0.0), mask=active)
            return 0

        lax.fori_loop(0, (mcount + 15) >> 4, post_body, 0)
        base = wid * per_worker + k * CHUNK
        pltpu.sync_copy(out_v.at[0], ro_hbm.at[pl.ds(base, CHUNK)])
        pltpu.sync_copy(out_v.at[1], go_hbm.at[pl.ds(base, CHUNK)])
        pltpu.sync_copy(out_v.at[2], bo_hbm.at[pl.ds(base, CHUNK)])
        pltpu.sync_copy(out_v.at[3], do_hbm.at[pl.ds(base, CHUNK)])

    def two_chunks(j, m_prev):
        k0 = 2 * j
        m0 = stage_a(k0, cidx_a, sub_a, pos_a, out_a)

        @pl.when(j > 0)
        def _():
            drain(m_prev, cidx_b, sem_b)
            stage_c(k0 - 1, m_prev, sub_b, pos_b, out_b)

        fire(m0, cidx_a, sem_a)
        m1 = stage_a(k0 + 1, cidx_b, sub_b, pos_b, out_b)
        drain(m0, cidx_a, sem_a)
        stage_c(k0, m0, sub_a, pos_a, out_a)
        fire(m1, cidx_b, sem_b)
        return m1

    m_last = lax.fori_loop(0, n_chunks // 2, two_chunks, jnp.int32(0))
    drain(m_last, cidx_b, sem_b)
    stage_c(n_chunks - 1, m_last, sub_b, pos_b, out_b)


@jax.jit
def _sc_voxels(xs, ys, zs, tab):
    n_pts = xs.shape[0]
    mesh = plsc.VectorSubcoreMesh(core_axis_name="c", subcore_axis_name="s")
    buf = lambda dt, *shape: pltpu.VMEM(tuple(shape), dt)
    return pl.kernel(
        _body,
        out_type=(
            jax.ShapeDtypeStruct((n_pts,), jnp.float32),
            jax.ShapeDtypeStruct((n_pts,), jnp.float32),
            jax.ShapeDtypeStruct((n_pts,), jnp.float32),
            jax.ShapeDtypeStruct((n_pts,), jnp.float32),
        ),
        mesh=mesh,
        compiler_params=pltpu.CompilerParams(
            needs_layout_passes=False, use_tc_tiling_on_sc=False),
        scratch_types=[
            buf(jnp.float32, CHUNK), buf(jnp.float32, CHUNK), buf(jnp.float32, CHUNK),
            buf(jnp.int32, 4 * RPC, W), buf(jnp.int32, CHUNK + 128),
            buf(jnp.int32, CHUNK + 128), buf(jnp.float32, 4, CHUNK),
            buf(jnp.int32, 4 * RPC, W), buf(jnp.int32, CHUNK + 128),
            buf(jnp.int32, CHUNK + 128), buf(jnp.float32, 4, CHUNK),
            buf(jnp.float32, 4 * RPC, W, 16),
            pltpu.SemaphoreType.DMA, pltpu.SemaphoreType.DMA,
        ],
    )(xs, ys, zs, tab)


def kernel(xyz, voxels):
    tab = (voxels.reshape(256, 256, 2, 128, 4)
           .transpose(0, 1, 2, 4, 3).reshape(1 << 22, 16))
    r, g, b, d = _sc_voxels(xyz[:, 0], xyz[:, 1], xyz[:, 2], tab)
    return jnp.stack([r, g, b], axis=1), d


# speedup vs baseline: 158.9554x; 1.3160x over previous
import jax
import jax.numpy as jnp
from jax import lax
from jax.experimental import pallas as pl
from jax.experimental.pallas import tpu as pltpu, tpu_sc as plsc

N_WORKERS = 32
CHUNK = 1024
W = 128
GROUPS = CHUNK // 16
BLOCKS = CHUNK // 128
RPC = CHUNK // W + 1


def _body(xs_hbm, ys_hbm, zs_hbm, tab_hbm, co_hbm, do_hbm,
          xin_a, cidx_a, sub_a, pos_a, out_a, dov_a,
          xin_b, cidx_b, sub_b, pos_b, out_b, dov_b,
          rows_v, sem_a, sem_b, sem_ia, sem_ib):
    wid = lax.axis_index("s") * 2 + lax.axis_index("c")
    n_pts = xs_hbm.shape[0]
    per_worker = n_pts // N_WORKERS
    n_chunks = per_worker // CHUNK

    ii = lax.iota(jnp.int32, 16)
    half16 = jnp.full((16,), 0.5, jnp.float32)
    zerof16 = jnp.zeros((16,), jnp.float32)
    zero16 = jnp.zeros((16,), jnp.int32)
    one16 = zero16 + 1
    two16 = zero16 + 2

    def fire_in(k, xin, sem):
        base = wid * per_worker + k * CHUNK
        pltpu.async_copy(xs_hbm.at[pl.ds(base, CHUNK)], xin.at[0], sem)
        pltpu.async_copy(ys_hbm.at[pl.ds(base, CHUNK)], xin.at[1], sem)
        pltpu.async_copy(zs_hbm.at[pl.ds(base, CHUNK)], xin.at[2], sem)

    def drain_in(k, xin, sem):
        base = wid * per_worker + k * CHUNK
        pltpu.make_async_copy(xs_hbm.at[pl.ds(base, CHUNK)], xin.at[0], sem).wait()
        pltpu.make_async_copy(ys_hbm.at[pl.ds(base, CHUNK)], xin.at[1], sem).wait()
        pltpu.make_async_copy(zs_hbm.at[pl.ds(base, CHUNK)], xin.at[2], sem).wait()

    def stage_a(k, xin, sem_in, kn, xin_n, sem_in_n,
                cidx_v, sub_v, pos_v, out_v, dov_v):
        drain_in(k, xin, sem_in)
        fire_in(kn, xin_n, sem_in_n)

        def index_body(g, mcount):
            o = g * 16
            x = xin[0, pl.ds(o, 16)]
            y = xin[1, pl.ds(o, 16)]
            z = xin[2, pl.ds(o, 16)]
            ux = jnp.clip(x * 128.0 + 128.0, 0.0, 255.0).astype(jnp.int32)
            uy = jnp.clip(y * 128.0 + 128.0, 0.0, 255.0).astype(jnp.int32)
            uz = jnp.clip(z * 128.0 + 128.0, 0.0, 255.0).astype(jnp.int32)
            zt = uz >> 7
            zl = uz & 127
            g0 = ((((ux << 8) | uy) << 1 | zt) << 5) | (zl >> 4)
            m = jnp.maximum(jnp.maximum(jnp.abs(x), jnp.abs(y)), jnp.abs(z))
            keep = m < 1.0
            tgt = mcount + plsc.cumsum(keep.astype(jnp.int32)) - 1
            row = tgt >> 7
            col = tgt & 127
            plsc.store_scatter(cidx_v, [row, col], g0, mask=keep)
            plsc.store_scatter(cidx_v, [RPC + row, col], g0 + 8, mask=keep)
            plsc.store_scatter(cidx_v, [2 * RPC + row, col], g0 + 16, mask=keep)
            plsc.store_scatter(cidx_v, [3 * RPC + row, col], g0 + 24, mask=keep)
            plsc.store_scatter(sub_v, [tgt], zl & 15, mask=keep)
            plsc.store_scatter(pos_v, [tgt], ii + o, mask=keep)
            nt = g // 8
            c0 = (g % 8) * 16
            out_v[nt, 0, pl.ds(c0, 16)] = half16
            out_v[nt, 1, pl.ds(c0, 16)] = half16
            out_v[nt, 2, pl.ds(c0, 16)] = half16
            dov_v[pl.ds(o, 16)] = zerof16
            return mcount + jnp.max(plsc.all_reduce_population_count(keep))

        mcount = lax.fori_loop(0, GROUPS, index_body, jnp.int32(0))

        for j in range(8):
            tgt = mcount + j * 16 + ii
            row = tgt >> 7
            col = tgt & 127
            dummy = ((wid << 8) | (j * 16 + ii)) << 5
            plsc.store_scatter(cidx_v, [row, col], dummy)
            plsc.store_scatter(cidx_v, [RPC + row, col], dummy)
            plsc.store_scatter(cidx_v, [2 * RPC + row, col], dummy)
            plsc.store_scatter(cidx_v, [3 * RPC + row, col], dummy)
        return mcount

    def fire(mcount, cidx_v, sem):
        def fire_body(r, _):
            for c in range(4):
                pltpu.async_copy(tab_hbm.at[cidx_v.at[c * RPC + r]],
                                 rows_v.at[c * RPC + r], sem)
            return 0
        lax.fori_loop(0, (mcount + 127) >> 7, fire_body, 0)

    def drain(mcount, cidx_v, sem):
        def drain_body(r, _):
            for c in range(4):
                pltpu.make_async_copy(tab_hbm.at[cidx_v.at[c * RPC + r]],
                                      rows_v.at[c * RPC + r], sem).wait()
            return 0
        lax.fori_loop(0, (mcount + 127) >> 7, drain_body, 0)

    def stage_c(k, mcount, sub_v, pos_v, out_v, dov_v):
        def post_body(t, _):
            o = t * 16
            s = o + ii
            active = s < mcount
            sub = sub_v[pl.ds(o, 16)]
            p = pos_v[pl.ds(o, 16)]
            row = s >> 7
            col = s & 127
            pt = p >> 7
            pc = p & 127
            vr = plsc.load_gather(rows_v, [row, col, sub], mask=active)
            vg = plsc.load_gather(rows_v, [RPC + row, col, sub], mask=active)
            vb = plsc.load_gather(rows_v, [2 * RPC + row, col, sub], mask=active)
            vd = plsc.load_gather(rows_v, [3 * RPC + row, col, sub], mask=active)
            plsc.store_scatter(out_v, [pt, zero16, pc],
                               1.0 / (1.0 + jnp.exp(-vr)), mask=active)
            plsc.store_scatter(out_v, [pt, one16, pc],
                               1.0 / (1.0 + jnp.exp(-vg)), mask=active)
            plsc.store_scatter(out_v, [pt, two16, pc],
                               1.0 / (1.0 + jnp.exp(-vb)), mask=active)
            plsc.store_scatter(dov_v, [p],
                               jnp.maximum(vd * 10.0, 0.0), mask=active)
            return 0

        lax.fori_loop(0, (mcount + 15) >> 4, post_body, 0)
        base = wid * per_worker + k * CHUNK
        pltpu.sync_copy(out_v, co_hbm.at[pl.ds(base // 128, BLOCKS)])
        pltpu.sync_copy(dov_v, do_hbm.at[pl.ds(base, CHUNK)])

    def two_chunks(j, m_prev):
        k0 = 2 * j
        m0 = stage_a(k0, xin_a, sem_ia, k0 + 1, xin_b, sem_ib,
                     cidx_a, sub_a, pos_a, out_a, dov_a)

        @pl.when(j > 0)
        def _():
            drain(m_prev, cidx_b, sem_b)
            stage_c(k0 - 1, m_prev, sub_b, pos_b, out_b, dov_b)

        fire(m0, cidx_a, sem_a)
        m1 = stage_a(k0 + 1, xin_b, sem_ib, (k0 + 2) % n_chunks, xin_a, sem_ia,
                     cidx_b, sub_b, pos_b, out_b, dov_b)
        drain(m0, cidx_a, sem_a)
        stage_c(k0, m0, sub_a, pos_a, out_a, dov_a)
        fire(m1, cidx_b, sem_b)
        return m1

    fire_in(0, xin_a, sem_ia)
    m_last = lax.fori_loop(0, n_chunks // 2, two_chunks, jnp.int32(0))
    drain(m_last, cidx_b, sem_b)
    stage_c(n_chunks - 1, m_last, sub_b, pos_b, out_b, dov_b)
    drain_in(0, xin_a, sem_ia)


@jax.jit
def _sc_voxels(xs, ys, zs, tab):
    n_pts = xs.shape[0]
    mesh = plsc.VectorSubcoreMesh(core_axis_name="c", subcore_axis_name="s")
    buf = lambda dt, *shape: pltpu.VMEM(tuple(shape), dt)
    pair = lambda: (
        buf(jnp.float32, 3, CHUNK),
        buf(jnp.int32, 4 * RPC, W),
        buf(jnp.int32, CHUNK + 128),
        buf(jnp.int32, CHUNK + 128),
        buf(jnp.float32, BLOCKS, 4, 128),
        buf(jnp.float32, CHUNK),
    )
    return pl.kernel(
        _body,
        out_type=(
            jax.ShapeDtypeStruct((n_pts // 128, 4, 128), jnp.float32),
            jax.ShapeDtypeStruct((n_pts,), jnp.float32),
        ),
        mesh=mesh,
        compiler_params=pltpu.CompilerParams(
            needs_layout_passes=False, use_tc_tiling_on_sc=False),
        scratch_types=[
            *pair(), *pair(),
            buf(jnp.float32, 4 * RPC, W, 16),
            pltpu.SemaphoreType.DMA, pltpu.SemaphoreType.DMA,
            pltpu.SemaphoreType.DMA, pltpu.SemaphoreType.DMA,
        ],
    )(xs, ys, zs, tab)


def kernel(xyz, voxels):
    n = xyz.shape[0]
    tab = (voxels.reshape(256, 256, 2, 128, 4)
           .transpose(0, 1, 2, 4, 3).reshape(1 << 22, 16))
    co, d = _sc_voxels(xyz[:, 0], xyz[:, 1], xyz[:, 2], tab)
    colors = co.transpose(0, 2, 1).reshape(n, 4)[:, :3]
    return colors, d
